# baseline (speedup 1.0000x reference)
import functools

import jax
import jax.numpy as jnp
from jax.experimental import pallas as pl
from jax.experimental.pallas import tpu as pltpu

N = 4096
DX = 512
HX = 1024
OX = 256
DA = 4096
HA = 1024
OA = 256
K = 10

MLP_BLOCK = 512
SIM_BLOCK = 256
TILE_ROWS = 64
CHUNK = 128


def _cmp_swap(a, b):
    if a is None:
        return b, None
    if b is None:
        return a, None
    return jnp.maximum(a, b), jnp.minimum(a, b)


def _oe_merge(a, b):
    n = len(a)
    if n == 1:
        hi, lo = _cmp_swap(a[0], b[0])
        return [hi, lo]
    ev = _oe_merge(a[0::2], b[0::2])
    od = _oe_merge(a[1::2], b[1::2])
    res = [ev[0]]
    for i in range(n - 1):
        hi, lo = _cmp_swap(od[i], ev[i + 1])
        res.extend([hi, lo])
    res.append(od[n - 1])
    return res


def _lane_topk(chunks, k):
    lists = [[c] for c in chunks]
    while len(lists) > 1:
        nxt = []
        for i in range(0, len(lists), 2):
            a = list(lists[i])
            b = list(lists[i + 1]) if i + 1 < len(lists) else []
            size = max(len(a), len(b), 1)
            size = 1 << (size - 1).bit_length()
            a += [None] * (size - len(a))
            b += [None] * (size - len(b))
            merged = _oe_merge(a, b)
            merged = [v for v in merged if v is not None][:k]
            nxt.append(merged)
        lists = nxt
    return lists[0]


def _mlp_kernel(x_ref, adj_ref, w1x_ref, b1x_ref, w2x_ref, b2x_ref,
                w1a_ref, b1a_ref, w2a_ref, b2a_ref,
                zx_ref, zxn_ref, za_ref, zan_ref):
    f32 = jnp.float32
    h1x = jnp.tanh(
        jax.lax.dot_general(x_ref[...], w1x_ref[...],
                            (((1,), (1,)), ((), ())),
                            preferred_element_type=f32) + b1x_ref[...])
    zx = jax.lax.dot_general(h1x, w2x_ref[...], (((1,), (1,)), ((), ())),
                             preferred_element_type=f32) + b2x_ref[...]
    zx_ref[...] = zx
    nx = jnp.sqrt(jnp.sum(zx * zx, axis=1, keepdims=True))
    zxn_ref[...] = zx / jnp.maximum(nx, 1e-12)

    h1a = jnp.tanh(
        jax.lax.dot_general(adj_ref[...], w1a_ref[...],
                            (((1,), (1,)), ((), ())),
                            preferred_element_type=f32) + b1a_ref[...])
    za = jax.lax.dot_general(h1a, w2a_ref[...], (((1,), (1,)), ((), ())),
                             preferred_element_type=f32) + b2a_ref[...]
    za_ref[...] = za
    na = jnp.sqrt(jnp.sum(za * za, axis=1, keepdims=True))
    zan_ref[...] = za / jnp.maximum(na, 1e-12)


def _sim_kernel(wa_ref, zx_blk_ref, za_blk_ref, zx_ref, za_ref,
                hx_ref, ha_ref, s_ref):
    f32 = jnp.float32
    hx = jax.lax.dot_general(zx_blk_ref[...], zx_ref[...],
                             (((1,), (1,)), ((), ())),
                             preferred_element_type=f32)
    ha = jax.lax.dot_general(za_blk_ref[...], za_ref[...],
                             (((1,), (1,)), ((), ())),
                             preferred_element_type=f32)
    hx_ref[...] = hx
    ha_ref[...] = ha

    ax = wa_ref[0] * (1.0 / jnp.maximum(
        jnp.sqrt(jnp.sum(hx * hx, axis=1, keepdims=True)), 1e-12))
    aa = wa_ref[1] * (1.0 / jnp.maximum(
        jnp.sqrt(jnp.sum(ha * ha, axis=1, keepdims=True)), 1e-12))

    neg_inf = f32(-jnp.inf)
    r0 = pl.program_id(0) * SIM_BLOCK
    nrows = TILE_ROWS

    for g in range(SIM_BLOCK // nrows):
        lo = g * nrows
        hxt = hx[lo:lo + nrows, :]
        hat = ha[lo:lo + nrows, :]
        st = ax[lo:lo + nrows, :] * hxt + aa[lo:lo + nrows, :] * hat
        cand = _lane_topk([st[:, i * CHUNK:(i + 1) * CHUNK]
                           for i in range(st.shape[1] // CHUNK)], K)
        m = jnp.max(cand[0], axis=1, keepdims=True)
        for j in range(2, K + 1):
            cur = jnp.where(cand[0] < m, cand[0], neg_inf)
            for i in range(1, j):
                cur = jnp.maximum(cur,
                                  jnp.where(cand[i] < m, cand[i], neg_inf))
            m = jnp.max(cur, axis=1, keepdims=True)
        s_ref[lo:lo + nrows, :] = (st >= m).astype(f32)

        g0 = r0 + lo
        w = (g0 // CHUNK) * CHUNK
        rows = jax.lax.broadcasted_iota(jnp.int32, (nrows, CHUNK), 0) + g0
        cols = jax.lax.broadcasted_iota(jnp.int32, (nrows, CHUNK), 1) + w
        slab = s_ref[lo:lo + nrows, pl.ds(w, CHUNK)]
        s_ref[lo:lo + nrows, pl.ds(w, CHUNK)] = slab \
            + (rows == cols).astype(f32)


@jax.jit
def kernel(x, adj, weights_a, W1x, b1x, W2x, b2x, W1a, b1a, W2a, b2a):
    f32 = jnp.float32
    wa = (weights_a / jnp.sum(weights_a)).astype(f32)

    nb = N // MLP_BLOCK
    row_blk = lambda i: (i, 0)
    whole = lambda i: (0, 0)
    mlp_out = pl.pallas_call(
        _mlp_kernel,
        grid=(nb,),
        in_specs=[
            pl.BlockSpec((MLP_BLOCK, DX), row_blk),
            pl.BlockSpec((MLP_BLOCK, DA), row_blk),
            pl.BlockSpec((HX, DX), whole),
            pl.BlockSpec((1, HX), whole),
            pl.BlockSpec((OX, HX), whole),
            pl.BlockSpec((1, OX), whole),
            pl.BlockSpec((HA, DA), whole),
            pl.BlockSpec((1, HA), whole),
            pl.BlockSpec((OA, HA), whole),
            pl.BlockSpec((1, OA), whole),
        ],
        out_specs=[
            pl.BlockSpec((MLP_BLOCK, OX), row_blk),
            pl.BlockSpec((MLP_BLOCK, OX), row_blk),
            pl.BlockSpec((MLP_BLOCK, OA), row_blk),
            pl.BlockSpec((MLP_BLOCK, OA), row_blk),
        ],
        out_shape=[
            jax.ShapeDtypeStruct((N, OX), f32),
            jax.ShapeDtypeStruct((N, OX), f32),
            jax.ShapeDtypeStruct((N, OA), f32),
            jax.ShapeDtypeStruct((N, OA), f32),
        ],
        compiler_params=pltpu.CompilerParams(
            dimension_semantics=("parallel",)),
    )(x, adj, W1x, b1x.reshape(1, HX), W2x, b2x.reshape(1, OX),
      W1a, b1a.reshape(1, HA), W2a, b2a.reshape(1, OA))
    zx, zx_norm, za, za_norm = mlp_out

    nsb = N // SIM_BLOCK
    sim_out = pl.pallas_call(
        _sim_kernel,
        grid=(nsb,),
        in_specs=[
            pl.BlockSpec(memory_space=pltpu.SMEM),
            pl.BlockSpec((SIM_BLOCK, OX), row_blk),
            pl.BlockSpec((SIM_BLOCK, OA), row_blk),
            pl.BlockSpec((N, OX), whole),
            pl.BlockSpec((N, OA), whole),
        ],
        out_specs=[
            pl.BlockSpec((SIM_BLOCK, N), row_blk),
            pl.BlockSpec((SIM_BLOCK, N), row_blk),
            pl.BlockSpec((SIM_BLOCK, N), row_blk),
        ],
        out_shape=[
            jax.ShapeDtypeStruct((N, N), f32),
            jax.ShapeDtypeStruct((N, N), f32),
            jax.ShapeDtypeStruct((N, N), f32),
        ],
        compiler_params=pltpu.CompilerParams(
            dimension_semantics=("parallel",)),
    )(wa, zx, za, zx, za)
    homo_x, homo_a, s_out = sim_out

    return (zx_norm, homo_x, za_norm, homo_a, s_out)


# speedup vs baseline: 1.0064x; 1.0064x over previous
import jax
import jax.numpy as jnp
from jax.experimental import pallas as pl
from jax.experimental.pallas import tpu as pltpu

N = 4096
DX = 512
HX = 1024
OX = 256
DA = 4096
HA = 1024
OA = 256
K = 10

MLP_BLOCK = 512
SIM_BLOCK = 256
TILE_ROWS = 32
CHUNK = 128


def _cmp_swap(a, b):
    if a is None:
        return b, None
    if b is None:
        return a, None
    return jnp.maximum(a, b), jnp.minimum(a, b)


def _oe_merge(a, b):
    n = len(a)
    if n == 1:
        hi, lo = _cmp_swap(a[0], b[0])
        return [hi, lo]
    ev = _oe_merge(a[0::2], b[0::2])
    od = _oe_merge(a[1::2], b[1::2])
    res = [ev[0]]
    for i in range(n - 1):
        hi, lo = _cmp_swap(od[i], ev[i + 1])
        res.extend([hi, lo])
    res.append(od[n - 1])
    return res


def _lane_topk(chunks, k):
    lists = [[c] for c in chunks]
    while len(lists) > 1:
        nxt = []
        for i in range(0, len(lists), 2):
            a = list(lists[i])
            b = list(lists[i + 1]) if i + 1 < len(lists) else []
            size = max(len(a), len(b), 1)
            size = 1 << (size - 1).bit_length()
            a += [None] * (size - len(a))
            b += [None] * (size - len(b))
            merged = _oe_merge(a, b)
            merged = [v for v in merged if v is not None][:k]
            nxt.append(merged)
        lists = nxt
    return lists[0]


def _mlp_kernel(x_ref, adj_ref, w1x_ref, b1x_ref, w2x_ref, b2x_ref,
                w1a_ref, b1a_ref, w2a_ref, b2a_ref,
                zx_ref, zxn_ref, za_ref, zan_ref):
    f32 = jnp.float32
    h1x = jnp.tanh(
        jax.lax.dot_general(x_ref[...], w1x_ref[...],
                            (((1,), (1,)), ((), ())),
                            preferred_element_type=f32) + b1x_ref[...])
    zx = jax.lax.dot_general(h1x, w2x_ref[...], (((1,), (1,)), ((), ())),
                             preferred_element_type=f32) + b2x_ref[...]
    zx_ref[...] = zx
    nx = jnp.sqrt(jnp.sum(zx * zx, axis=1, keepdims=True))
    zxn_ref[...] = zx / jnp.maximum(nx, 1e-12)

    h1a = jnp.tanh(
        jax.lax.dot_general(adj_ref[...], w1a_ref[...],
                            (((1,), (1,)), ((), ())),
                            preferred_element_type=f32) + b1a_ref[...])
    za = jax.lax.dot_general(h1a, w2a_ref[...], (((1,), (1,)), ((), ())),
                             preferred_element_type=f32) + b2a_ref[...]
    za_ref[...] = za
    na = jnp.sqrt(jnp.sum(za * za, axis=1, keepdims=True))
    zan_ref[...] = za / jnp.maximum(na, 1e-12)


def _sim_kernel(wa_ref, zx_blk_ref, za_blk_ref, zx_ref, za_ref,
                hx_ref, ha_ref, s_ref):
    f32 = jnp.float32
    hx = jax.lax.dot_general(zx_blk_ref[...], zx_ref[...],
                             (((1,), (1,)), ((), ())),
                             preferred_element_type=f32)
    ha = jax.lax.dot_general(za_blk_ref[...], za_ref[...],
                             (((1,), (1,)), ((), ())),
                             preferred_element_type=f32)
    hx_ref[...] = hx
    ha_ref[...] = ha

    ax = wa_ref[0] * (1.0 / jnp.maximum(
        jnp.sqrt(jnp.sum(hx * hx, axis=1, keepdims=True)), 1e-12))
    aa = wa_ref[1] * (1.0 / jnp.maximum(
        jnp.sqrt(jnp.sum(ha * ha, axis=1, keepdims=True)), 1e-12))

    neg_inf = f32(-jnp.inf)
    r0 = pl.program_id(0) * SIM_BLOCK
    nrows = TILE_ROWS

    for g in range(SIM_BLOCK // nrows):
        lo = g * nrows
        hxt = hx[lo:lo + nrows, :]
        hat = ha[lo:lo + nrows, :]
        st = ax[lo:lo + nrows, :] * hxt + aa[lo:lo + nrows, :] * hat
        cand = _lane_topk([st[:, i * CHUNK:(i + 1) * CHUNK]
                           for i in range(st.shape[1] // CHUNK)], K)
        m = jnp.max(cand[0], axis=1, keepdims=True)
        for j in range(2, K + 1):
            cur = jnp.where(cand[0] < m, cand[0], neg_inf)
            for i in range(1, j):
                cur = jnp.maximum(cur,
                                  jnp.where(cand[i] < m, cand[i], neg_inf))
            m = jnp.max(cur, axis=1, keepdims=True)
        s_ref[lo:lo + nrows, :] = (st >= m).astype(f32)

        g0 = r0 + lo
        w = (g0 // CHUNK) * CHUNK
        rows = jax.lax.broadcasted_iota(jnp.int32, (nrows, CHUNK), 0) + g0
        cols = jax.lax.broadcasted_iota(jnp.int32, (nrows, CHUNK), 1) + w
        slab = s_ref[lo:lo + nrows, pl.ds(w, CHUNK)]
        s_ref[lo:lo + nrows, pl.ds(w, CHUNK)] = slab \
            + (rows == cols).astype(f32)


@jax.jit
def kernel(x, adj, weights_a, W1x, b1x, W2x, b2x, W1a, b1a, W2a, b2a):
    f32 = jnp.float32
    wa = (weights_a / jnp.sum(weights_a)).astype(f32)

    nb = N // MLP_BLOCK
    row_blk = lambda i: (i, 0)
    whole = lambda i: (0, 0)
    mlp_out = pl.pallas_call(
        _mlp_kernel,
        grid=(nb,),
        in_specs=[
            pl.BlockSpec((MLP_BLOCK, DX), row_blk),
            pl.BlockSpec((MLP_BLOCK, DA), row_blk),
            pl.BlockSpec((HX, DX), whole),
            pl.BlockSpec((1, HX), whole),
            pl.BlockSpec((OX, HX), whole),
            pl.BlockSpec((1, OX), whole),
            pl.BlockSpec((HA, DA), whole),
            pl.BlockSpec((1, HA), whole),
            pl.BlockSpec((OA, HA), whole),
            pl.BlockSpec((1, OA), whole),
        ],
        out_specs=[
            pl.BlockSpec((MLP_BLOCK, OX), row_blk),
            pl.BlockSpec((MLP_BLOCK, OX), row_blk),
            pl.BlockSpec((MLP_BLOCK, OA), row_blk),
            pl.BlockSpec((MLP_BLOCK, OA), row_blk),
        ],
        out_shape=[
            jax.ShapeDtypeStruct((N, OX), f32),
            jax.ShapeDtypeStruct((N, OX), f32),
            jax.ShapeDtypeStruct((N, OA), f32),
            jax.ShapeDtypeStruct((N, OA), f32),
        ],
        compiler_params=pltpu.CompilerParams(
            dimension_semantics=("parallel",)),
    )(x, adj, W1x, b1x.reshape(1, HX), W2x, b2x.reshape(1, OX),
      W1a, b1a.reshape(1, HA), W2a, b2a.reshape(1, OA))
    zx, zx_norm, za, za_norm = mlp_out

    nsb = N // SIM_BLOCK
    sim_out = pl.pallas_call(
        _sim_kernel,
        grid=(nsb,),
        in_specs=[
            pl.BlockSpec(memory_space=pltpu.SMEM),
            pl.BlockSpec((SIM_BLOCK, OX), row_blk),
            pl.BlockSpec((SIM_BLOCK, OA), row_blk),
            pl.BlockSpec((N, OX), whole),
            pl.BlockSpec((N, OA), whole),
        ],
        out_specs=[
            pl.BlockSpec((SIM_BLOCK, N), row_blk),
            pl.BlockSpec((SIM_BLOCK, N), row_blk),
            pl.BlockSpec((SIM_BLOCK, N), row_blk),
        ],
        out_shape=[
            jax.ShapeDtypeStruct((N, N), f32),
            jax.ShapeDtypeStruct((N, N), f32),
            jax.ShapeDtypeStruct((N, N), f32),
        ],
        compiler_params=pltpu.CompilerParams(
            dimension_semantics=("parallel",)),
    )(wa, zx, za, zx, za)
    homo_x, homo_a, s_out = sim_out

    return (zx_norm, homo_x, za_norm, homo_a, s_out)
